# baseline (speedup 1.0000x reference)
import jax
import jax.numpy as jnp
from jax import lax
from jax.experimental import pallas as pl
from jax.experimental.pallas import tpu as pltpu, tpu_sc as plsc

WIDTH = 1000
FEATURE_DIM = 100000
N = 1024
NW = 32
ROWS_PER_W = N // NW
SLICE = ROWS_PER_W * FEATURE_DIM

_GATHER_DN = jax.lax.GatherDimensionNumbers(
    offset_dims=(), collapsed_slice_dims=(0,), start_index_map=(0,))


def _gather16(v, g):
    return jax.lax.gather(
        v, g[:, None], _GATHER_DN, (1,),
        mode=jax.lax.GatherScatterMode.PROMISE_IN_BOUNDS)


def _sc_body(zeros_hbm, state_hbm, out_hbm, state_v, idx_v, ones_v, sem, zsem):
    c = lax.axis_index("c")
    s = lax.axis_index("s")
    wid = s * 2 + c
    base_row = wid * ROWS_PER_W
    base_flat = base_row * FEATURE_DIM

    pltpu.sync_copy(state_hbm.at[pl.ds(2 * base_row, 2 * ROWS_PER_W)], state_v)

    lane = lax.broadcasted_iota(jnp.int32, (16,), 0)
    even = (2 * lane) % 16
    odd = (2 * lane + 1) % 16
    one16 = jnp.ones((16,), jnp.float32)
    for g in range(ROWS_PER_W // 8):
        v = state_v[pl.ds(g * 16, 16)]
        xs = _gather16(v, even)
        ys = _gather16(v, odd)
        rows = g * 8 + (lane % 8)
        idx_v[pl.ds(g * 16, 16)] = (
            base_flat + rows * FEATURE_DIM + xs + WIDTH * ys)
        ones_v[pl.ds(g * 16, 16)] = one16

    pltpu.make_async_copy(
        zeros_hbm, out_hbm.at[pl.ds(base_flat, SLICE)], zsem).start()
    pltpu.make_async_copy(
        zeros_hbm, out_hbm.at[pl.ds(base_flat, SLICE)], zsem).wait()

    pltpu.async_copy(ones_v, out_hbm.at[idx_v], sem).wait()


def kernel(state):
    n = state.shape[0]
    zeros_flat = jnp.zeros((SLICE,), jnp.float32)
    out = pl.kernel(
        _sc_body,
        out_type=jax.ShapeDtypeStruct((n * FEATURE_DIM,), jnp.float32),
        mesh=plsc.VectorSubcoreMesh(core_axis_name="c", subcore_axis_name="s"),
        scratch_types=[
            pltpu.VMEM((2 * ROWS_PER_W,), jnp.int32),
            pltpu.VMEM((2 * ROWS_PER_W,), jnp.int32),
            pltpu.VMEM((2 * ROWS_PER_W,), jnp.float32),
            pltpu.SemaphoreType.DMA,
            pltpu.SemaphoreType.DMA,
        ],
    )(zeros_flat, state.reshape(-1))
    return out.reshape(n, FEATURE_DIM)


# speedup vs baseline: 27.9065x; 27.9065x over previous
import jax
import jax.numpy as jnp
from jax.experimental import pallas as pl

WIDTH = 1000
FEATURE_DIM = 100000
ROW_BLOCK = 16


def _onehot_block(state_ref, out_ref):
    idx = state_ref[:, 0] + WIDTH * state_ref[:, 1]
    cols = jax.lax.broadcasted_iota(jnp.int32, out_ref.shape, 1)
    out_ref[...] = (cols == idx[:, None]).astype(jnp.float32)


def kernel(state):
    n = state.shape[0]
    grid = n // ROW_BLOCK
    return pl.pallas_call(
        _onehot_block,
        grid=(grid,),
        in_specs=[pl.BlockSpec((ROW_BLOCK, 2), lambda i: (i, 0))],
        out_specs=pl.BlockSpec((ROW_BLOCK, FEATURE_DIM), lambda i: (i, 0)),
        out_shape=jax.ShapeDtypeStruct((n, FEATURE_DIM), jnp.float32),
    )(state)
